# Initial kernel scaffold; baseline (speedup 1.0000x reference)
import jax
import jax.numpy as jnp
from jax.experimental import pallas as pl


def kernel(x, w_router, w_gate, w_up, w_down):
    raise NotImplementedError("write your pallas kernel here")


# speedup vs baseline: 1.3149x; 1.3149x over previous
import functools

import jax
import jax.numpy as jnp
from jax.experimental import pallas as pl
from jax.experimental.pallas import tpu as pltpu

T, D, DFF, E, K = 2048, 1024, 2048, 8, 2
TB = 256
FC = 1024
NF = DFF // FC
NT = T // TB


def _moe_body(x_ref, wr_ref, wg_ref, wu_ref, wd_ref, out_ref):
    e = pl.program_id(0)
    f = pl.program_id(1)
    t = pl.program_id(2)

    x = x_ref[...]
    logits = jax.lax.dot_general(x, wr_ref[...],
                                 (((1,), (1,)), ((), ())),
                                 preferred_element_type=jnp.float32)
    iota_e = jax.lax.broadcasted_iota(jnp.int32, (TB, E), 1)
    m1 = jnp.max(logits, axis=1, keepdims=True)
    i1 = jnp.min(jnp.where(logits == m1, iota_e, E), axis=1, keepdims=True)
    masked = jnp.where(iota_e == i1, -jnp.inf, logits)
    m2 = jnp.max(masked, axis=1, keepdims=True)
    i2 = jnp.min(jnp.where(masked == m2, iota_e, E), axis=1, keepdims=True)
    d = jnp.exp(m2 - m1)
    g1 = 1.0 / (1.0 + d)
    g2 = d / (1.0 + d)
    w = jnp.where(i1 == e, g1, 0.0) + jnp.where(i2 == e, g2, 0.0)

    wg = wg_ref[0]
    wu = wu_ref[0]
    wd = wd_ref[0]
    g = jax.lax.dot_general(x, wg, (((1,), (1,)), ((), ())),
                            preferred_element_type=jnp.float32)
    u = jax.lax.dot_general(x, wu, (((1,), (1,)), ((), ())),
                            preferred_element_type=jnp.float32)
    h = (g * (1.0 / (1.0 + jnp.exp(-g)))) * u
    y = jax.lax.dot_general(h, wd, (((1,), (1,)), ((), ())),
                            preferred_element_type=jnp.float32)
    y = y * w

    first = jnp.logical_and(e == 0, f == 0)

    @pl.when(first)
    def _():
        out_ref[pl.ds(t * TB, TB), :] = y

    @pl.when(jnp.logical_not(first))
    def _():
        out_ref[pl.ds(t * TB, TB), :] += y


@jax.jit
def kernel(x, w_router, w_gate, w_up, w_down):
    return pl.pallas_call(
        _moe_body,
        grid=(E, NF, NT),
        in_specs=[
            pl.BlockSpec((TB, D), lambda e, f, t: (t, 0)),
            pl.BlockSpec((E, D), lambda e, f, t: (0, 0)),
            pl.BlockSpec((1, FC, D), lambda e, f, t: (e, f, 0)),
            pl.BlockSpec((1, FC, D), lambda e, f, t: (e, f, 0)),
            pl.BlockSpec((1, D, FC), lambda e, f, t: (e, 0, f)),
        ],
        out_specs=pl.BlockSpec((T, D), lambda e, f, t: (0, 0)),
        out_shape=jax.ShapeDtypeStruct((T, D), jnp.float32),
    )(x, w_router, w_gate, w_up, w_down)
